# baseline (speedup 1.0000x reference)
import jax
import jax.numpy as jnp
from jax.experimental import pallas as pl
from jax.experimental.pallas import tpu as pltpu

N = 1024
D_IN = 512
D_OUT = 64
A_CHUNKS = 4
AC = N // A_CHUNKS


def _gcn_body(x_hbm, a_hbm, w_hbm, b_ref, o_ref, xv, av, wv, xsem, asem, wsem):
    cw = pltpu.make_async_copy(w_hbm, wv, wsem)
    cx = pltpu.make_async_copy(x_hbm, xv, xsem)
    cw.start()
    cx.start()
    a_copies = [
        pltpu.make_async_copy(
            a_hbm.at[pl.ds(i * AC, AC), :], av.at[pl.ds(i * AC, AC), :], asem.at[i]
        )
        for i in range(A_CHUNKS)
    ]
    for c in a_copies:
        c.start()
    cw.wait()
    cx.wait()
    sup = jnp.dot(xv[:], wv[:], preferred_element_type=jnp.float32)
    for i in range(A_CHUNKS):
        a_copies[i].wait()
        o_ref[pl.ds(i * AC, AC), :] = (
            jnp.dot(av[pl.ds(i * AC, AC), :], sup, preferred_element_type=jnp.float32)
            + b_ref[:]
        )


def kernel(input, adj, weight, bias):
    return pl.pallas_call(
        _gcn_body,
        in_specs=[
            pl.BlockSpec(memory_space=pl.ANY),
            pl.BlockSpec(memory_space=pl.ANY),
            pl.BlockSpec(memory_space=pl.ANY),
            pl.BlockSpec(memory_space=pltpu.VMEM),
        ],
        out_specs=pl.BlockSpec(memory_space=pltpu.VMEM),
        out_shape=jax.ShapeDtypeStruct((N, D_OUT), jnp.float32),
        scratch_shapes=[
            pltpu.VMEM((N, D_IN), jnp.float32),
            pltpu.VMEM((N, N), jnp.float32),
            pltpu.VMEM((D_IN, D_OUT), jnp.float32),
            pltpu.SemaphoreType.DMA,
            pltpu.SemaphoreType.DMA((A_CHUNKS,)),
            pltpu.SemaphoreType.DMA,
        ],
    )(input, adj, weight, bias.reshape(1, D_OUT))


# speedup vs baseline: 1.1563x; 1.1563x over previous
import jax
import jax.numpy as jnp
from jax.experimental import pallas as pl

N = 1024
D_IN = 512
D_OUT = 64


def _gcn_body(x_ref, a_ref, w_ref, b_ref, o_ref):
    sup = jnp.dot(x_ref[:], w_ref[:], preferred_element_type=jnp.float32,
                  precision=jax.lax.Precision.DEFAULT)
    o_ref[:] = jnp.dot(a_ref[:], sup, preferred_element_type=jnp.float32,
                       precision=jax.lax.Precision.DEFAULT) + b_ref[:]


def kernel(input, adj, weight, bias):
    return pl.pallas_call(
        _gcn_body,
        out_shape=jax.ShapeDtypeStruct((N, D_OUT), jnp.float32),
    )(input, adj, weight, bias.reshape(1, D_OUT))
